# baseline (speedup 1.0000x reference)
import math

import jax
import jax.numpy as jnp
from jax.experimental import pallas as pl
from jax.experimental.pallas import tpu as pltpu

EMBED = 1024
NH = 16
DH = 64
BLK = 32
STR = 16
SELB = 64
NSEL = 16
WIN = 512
KNN = 8

SQ = 256


def _softmax_last(s):
    m = jnp.max(s, axis=-1, keepdims=True)
    e = jnp.exp(s - m)
    return e / jnp.sum(e, axis=-1, keepdims=True)


def _compress1_body(v_ref, p_ref, w1v_ref, w1p_ref, b1_ref, hs_ref):
    z = jnp.dot(v_ref[...], w1v_ref[...], preferred_element_type=jnp.float32)
    z = z + jnp.dot(p_ref[...], w1p_ref[...], preferred_element_type=jnp.float32)
    z = z + b1_ref[...]
    r = jnp.maximum(z, 0.0)
    hs_ref[...] = r.reshape(8, 16, EMBED).sum(axis=1)


def _compress2_body(hs_ref, w2_ref, b2_ref, out_ref):
    hs = hs_ref[0]
    bm = (hs + jnp.roll(hs, -1, axis=0)) * (1.0 / BLK)
    out_ref[0] = jnp.dot(bm, w2_ref[...], preferred_element_type=jnp.float32) + b2_ref[...]


def _attn_gate_body(top_ref, q_ref, k_ref, v_ref, ck_ref, cv_ref,
                    qm_ref, wgc_ref, wgs_ref, wgw_ref, bg_ref, o_ref,
                    selk_ref, selv_ref):
    scale = 1.0 / math.sqrt(DH)
    b = pl.program_id(0)
    sstep = pl.program_id(1)
    S = k_ref.shape[1]

    @pl.when(sstep == 0)
    def _assemble():
        for j in range(NSEL):
            t = top_ref[b, j]
            base = jnp.where(t < S // SELB, t * SELB, S - SELB)
            dup = t >= S // SELB
            kblk = k_ref[0, pl.ds(base, SELB), :]
            vblk = v_ref[0, pl.ds(base, SELB), :]
            last_k = jnp.broadcast_to(kblk[SELB - 1:SELB, :], (SELB, EMBED))
            last_v = jnp.broadcast_to(vblk[SELB - 1:SELB, :], (SELB, EMBED))
            selk_ref[pl.ds(j * SELB, SELB), :] = jnp.where(dup, last_k, kblk)
            selv_ref[pl.ds(j * SELB, SELB), :] = jnp.where(dup, last_v, vblk).astype(jnp.bfloat16)

    qb = q_ref[0]
    selk = selk_ref[...]
    selv = selv_ref[...]
    kw = k_ref[0, S - WIN:, :]
    vw = v_ref[0, S - WIN:, :].astype(jnp.bfloat16)
    ck = ck_ref[0]
    cv = cv_ref[0].astype(jnp.bfloat16)
    qm = qm_ref[0]
    nbmask = jax.lax.broadcasted_iota(jnp.int32, (1, 128), 1) < 127

    slc_parts, win_parts, cmp_parts = [], [], []
    dn = (((1,), (1,)), ((), ()))
    for h in range(NH):
        sl = slice(h * DH, (h + 1) * DH)
        qh = qb[:, sl]
        s1 = jax.lax.dot_general(qh, selk[:, sl], dn,
                                 preferred_element_type=jnp.float32) * scale
        slc_parts.append(jnp.dot(_softmax_last(s1).astype(jnp.bfloat16), selv[:, sl],
                                 preferred_element_type=jnp.float32))
        s2 = jax.lax.dot_general(qh, kw[:, sl], dn,
                                 preferred_element_type=jnp.float32) * scale
        win_parts.append(jnp.dot(_softmax_last(s2).astype(jnp.bfloat16), vw[:, sl],
                                 preferred_element_type=jnp.float32))
        s3 = jax.lax.dot_general(qm[:, sl], ck[:, sl], dn,
                                 preferred_element_type=jnp.float32) * scale
        s3 = jnp.where(nbmask, s3, -1e30)
        cmp_parts.append(jnp.dot(_softmax_last(s3).astype(jnp.bfloat16), cv[:, sl],
                                 preferred_element_type=jnp.float32))
    t_slc = jnp.concatenate(slc_parts, axis=-1)
    t_win = jnp.concatenate(win_parts, axis=-1)
    t_cmp = jnp.concatenate(cmp_parts, axis=-1)

    logits = []
    for j in range(3):
        lj = (jnp.sum(t_cmp * wgc_ref[j:j + 1, :], axis=1, keepdims=True)
              + jnp.sum(t_slc * wgs_ref[j:j + 1, :], axis=1, keepdims=True)
              + jnp.sum(t_win * wgw_ref[j:j + 1, :], axis=1, keepdims=True)
              + bg_ref[j])
        logits.append(lj)
    m = jnp.maximum(jnp.maximum(logits[0], logits[1]), logits[2])
    e0, e1, e2 = (jnp.exp(l - m) for l in logits)
    den = e0 + e1 + e2
    o_ref[0] = (e0 / den) * t_cmp + (e1 / den) * t_slc + (e2 / den) * t_win


def kernel(x, positions, W_qkv, b_qkv, W_c1, b_c1, W_c2, b_c2, W_g, b_g):
    S, B, D = x.shape
    H, dh = NH, DH
    nb = math.ceil((S - BLK) / STR) + 1

    qkv = (x @ W_qkv + b_qkv).reshape(S, B, H, 3 * dh).transpose(1, 2, 0, 3)
    q, k, v = jnp.split(qkv, 3, axis=-1)
    starts = jnp.arange(nb) * STR
    blk_idx = starts[:, None] + jnp.arange(BLK)
    k_tok = k.transpose(2, 0, 1, 3).reshape(S, B, D)
    v_tok = v.transpose(2, 0, 1, 3).reshape(S, B, D)
    pb = positions[blk_idx]

    blocks = k_tok[blk_idx]
    hk = jnp.concatenate([blocks, pb], axis=-1)
    hk = jax.nn.relu(hk @ W_c1 + b_c1) @ W_c2 + b_c2
    hk = hk.mean(axis=1)
    cmp_k = hk.reshape(nb, B, H, dh).transpose(1, 2, 0, 3)

    imp = jax.nn.softmax(
        jnp.matmul(q.mean(axis=1), jnp.swapaxes(cmp_k.mean(axis=1), -2, -1))
        / math.sqrt(dh), axis=-1).mean(axis=-1)
    _, top_blocks = jax.lax.top_k(imp, NSEL)

    k_bt = k_tok.transpose(1, 0, 2)
    v_bt = v_tok.transpose(1, 0, 2)
    cmpk_tok = cmp_k.transpose(0, 2, 1, 3).reshape(B, nb, D)
    cmpk_tok = jnp.concatenate(
        [cmpk_tok, jnp.zeros((B, 128 - nb, D), jnp.float32)], axis=1)
    qmean = q.mean(axis=2).reshape(B, 1, D)

    pos_pad = jnp.zeros((B * S, 128), jnp.float32).at[:, :3].set(
        positions.transpose(1, 0, 2).reshape(B * S, 3))
    w1v = W_c1[:D]
    w1p = jnp.zeros((128, D), jnp.float32).at[:3].set(W_c1[D:])

    halfs = pl.pallas_call(
        _compress1_body,
        grid=(B * S // 128,),
        in_specs=[
            pl.BlockSpec((128, D), lambda i: (i, 0)),
            pl.BlockSpec((128, 128), lambda i: (i, 0)),
            pl.BlockSpec((D, D), lambda i: (0, 0)),
            pl.BlockSpec((128, D), lambda i: (0, 0)),
            pl.BlockSpec((1, D), lambda i: (0, 0)),
        ],
        out_specs=pl.BlockSpec((8, D), lambda i: (i, 0)),
        out_shape=jax.ShapeDtypeStruct((B * S // 16, D), jnp.float32),
    )(v_bt.reshape(B * S, D), pos_pad, w1v, w1p, b_c1.reshape(1, D))

    cmp_v_tok = pl.pallas_call(
        _compress2_body,
        grid=(B,),
        in_specs=[
            pl.BlockSpec((1, 128, D), lambda b: (b, 0, 0)),
            pl.BlockSpec((D, D), lambda b: (0, 0)),
            pl.BlockSpec((1, D), lambda b: (0, 0)),
        ],
        out_specs=pl.BlockSpec((1, 128, D), lambda b: (b, 0, 0)),
        out_shape=jax.ShapeDtypeStruct((B, 128, D), jnp.float32),
    )(halfs.reshape(B, 128, D), W_c2, b_c2.reshape(1, D))

    wg_c = jnp.zeros((8, D), jnp.float32).at[:3].set(W_g[0:D].T)
    wg_s = jnp.zeros((8, D), jnp.float32).at[:3].set(W_g[D:2 * D].T)
    wg_w = jnp.zeros((8, D), jnp.float32).at[:3].set(W_g[2 * D:].T)

    out = pl.pallas_call(
        _attn_gate_body,
        grid=(B, S // SQ),
        in_specs=[
            pl.BlockSpec(memory_space=pltpu.SMEM),
            pl.BlockSpec((1, SQ, D), lambda b, s: (b, s, 0)),
            pl.BlockSpec((1, S, D), lambda b, s: (b, 0, 0)),
            pl.BlockSpec((1, S, D), lambda b, s: (b, 0, 0)),
            pl.BlockSpec((1, 128, D), lambda b, s: (b, 0, 0)),
            pl.BlockSpec((1, 128, D), lambda b, s: (b, 0, 0)),
            pl.BlockSpec((1, 1, D), lambda b, s: (b, 0, 0)),
            pl.BlockSpec((8, D), lambda b, s: (0, 0)),
            pl.BlockSpec((8, D), lambda b, s: (0, 0)),
            pl.BlockSpec((8, D), lambda b, s: (0, 0)),
            pl.BlockSpec(memory_space=pltpu.SMEM),
        ],
        out_specs=pl.BlockSpec((1, SQ, D), lambda b, s: (b, s, 0)),
        out_shape=jax.ShapeDtypeStruct((B, S, D), jnp.float32),
        scratch_shapes=[
            pltpu.VMEM((NSEL * SELB, D), jnp.float32),
            pltpu.VMEM((NSEL * SELB, D), jnp.bfloat16),
        ],
    )(top_blocks, q.transpose(0, 2, 1, 3).reshape(B, S, D), k_bt, v_bt,
      cmpk_tok, cmp_v_tok, qmean, wg_c, wg_s, wg_w, b_g)

    return out


# speedup vs baseline: 1.3863x; 1.3863x over previous
import math

import jax
import jax.numpy as jnp
from jax.experimental import pallas as pl
from jax.experimental.pallas import tpu as pltpu

EMBED = 1024
NH = 16
DH = 64
BLK = 32
STR = 16
SELB = 64
NSEL = 16
WIN = 512
KNN = 8

SQ = 256


def _softmax_last(s):
    m = jnp.max(s, axis=-1, keepdims=True)
    e = jnp.exp(s - m)
    return e / jnp.sum(e, axis=-1, keepdims=True)


def _compress1_body(v_ref, p_ref, w1v_ref, w1p_ref, b1_ref, hs_ref):
    z = jnp.dot(v_ref[...], w1v_ref[...], preferred_element_type=jnp.float32)
    z = z + jnp.dot(p_ref[...], w1p_ref[...], preferred_element_type=jnp.float32)
    z = z + b1_ref[...]
    r = jnp.maximum(z, 0.0)
    hs_ref[...] = r.reshape(8, 16, EMBED).sum(axis=1)


def _compress2_body(hs_ref, w2_ref, b2_ref, out_ref):
    hs = hs_ref[0]
    bm = (hs + jnp.roll(hs, -1, axis=0)) * (1.0 / BLK)
    out_ref[0] = jnp.dot(bm, w2_ref[...], preferred_element_type=jnp.float32) + b2_ref[...]


def _sel_assemble_body(top_ref, k_ref, v_ref, selk_ref, selv_ref):
    b = pl.program_id(0)
    S = k_ref.shape[1]
    for j in range(NSEL):
        t = top_ref[b, j]
        base = jnp.where(t < S // SELB, t * SELB, S - SELB)
        dup = t >= S // SELB
        kblk = k_ref[0, pl.ds(base, SELB), :]
        vblk = v_ref[0, pl.ds(base, SELB), :]
        last_k = jnp.broadcast_to(kblk[SELB - 1:SELB, :], (SELB, EMBED))
        last_v = jnp.broadcast_to(vblk[SELB - 1:SELB, :], (SELB, EMBED))
        selk_ref[0, pl.ds(j * SELB, SELB), :] = jnp.where(dup, last_k, kblk)
        selv_ref[0, pl.ds(j * SELB, SELB), :] = jnp.where(dup, last_v, vblk)


def _attn_gate_body(q_ref, selk_ref, selv_ref, kw_ref, vw_ref, ck_ref, cv_ref,
                    qm_ref, wgc_ref, wgs_ref, wgw_ref, bg_ref, o_ref):
    scale = 1.0 / math.sqrt(DH)

    qb = q_ref[0] * scale
    selk = selk_ref[0]
    selv = selv_ref[0]
    kw = kw_ref[0]
    vw = vw_ref[0]
    ck = ck_ref[0]
    cv = cv_ref[0]
    qm = qm_ref[0] * scale
    nbmask = jax.lax.broadcasted_iota(jnp.int32, (1, 128), 1) < 127

    def _attend(qh, kh, vh):
        s = jax.lax.dot_general(qh, kh, (((1,), (1,)), ((), ())),
                                preferred_element_type=jnp.float32)
        e = jnp.exp(s)
        o = jnp.dot(e, vh, preferred_element_type=jnp.float32)
        return o / jnp.sum(e, axis=-1, keepdims=True)

    slc_parts, win_parts, cmp_parts = [], [], []
    dn = (((1,), (1,)), ((), ()))
    for h in range(NH):
        sl = slice(h * DH, (h + 1) * DH)
        qh = qb[:, sl]
        slc_parts.append(_attend(qh, selk[:, sl], selv[:, sl]))
        win_parts.append(_attend(qh, kw[:, sl], vw[:, sl]))
        s3 = jax.lax.dot_general(qm[:, sl], ck[:, sl], dn,
                                 preferred_element_type=jnp.float32)
        e3 = jnp.where(nbmask, jnp.exp(s3), 0.0)
        o3 = jnp.dot(e3, cv[:, sl], preferred_element_type=jnp.float32)
        cmp_parts.append(o3 / jnp.sum(e3, axis=-1, keepdims=True))
    t_slc = jnp.concatenate(slc_parts, axis=-1)
    t_win = jnp.concatenate(win_parts, axis=-1)
    t_cmp = jnp.concatenate(cmp_parts, axis=-1)

    logits = []
    for j in range(3):
        lj = (jnp.sum(t_cmp * wgc_ref[j:j + 1, :], axis=1, keepdims=True)
              + jnp.sum(t_slc * wgs_ref[j:j + 1, :], axis=1, keepdims=True)
              + jnp.sum(t_win * wgw_ref[j:j + 1, :], axis=1, keepdims=True)
              + bg_ref[j])
        logits.append(lj)
    m = jnp.maximum(jnp.maximum(logits[0], logits[1]), logits[2])
    e0, e1, e2 = (jnp.exp(l - m) for l in logits)
    den = e0 + e1 + e2
    o_ref[0] = (e0 / den) * t_cmp + (e1 / den) * t_slc + (e2 / den) * t_win


def kernel(x, positions, W_qkv, b_qkv, W_c1, b_c1, W_c2, b_c2, W_g, b_g):
    S, B, D = x.shape
    H, dh = NH, DH
    nb = math.ceil((S - BLK) / STR) + 1

    qkv = (x @ W_qkv + b_qkv).reshape(S, B, H, 3 * dh).transpose(1, 2, 0, 3)
    q, k, v = jnp.split(qkv, 3, axis=-1)
    starts = jnp.arange(nb) * STR
    blk_idx = starts[:, None] + jnp.arange(BLK)
    k_tok = k.transpose(2, 0, 1, 3).reshape(S, B, D)
    v_tok = v.transpose(2, 0, 1, 3).reshape(S, B, D)
    pb = positions[blk_idx]

    blocks = k_tok[blk_idx]
    hk = jnp.concatenate([blocks, pb], axis=-1)
    hk = jax.nn.relu(hk @ W_c1 + b_c1) @ W_c2 + b_c2
    hk = hk.mean(axis=1)
    cmp_k = hk.reshape(nb, B, H, dh).transpose(1, 2, 0, 3)

    imp = jax.nn.softmax(
        jnp.matmul(q.mean(axis=1), jnp.swapaxes(cmp_k.mean(axis=1), -2, -1))
        / math.sqrt(dh), axis=-1).mean(axis=-1)
    _, top_blocks = jax.lax.top_k(imp, NSEL)

    k_bt = k_tok.transpose(1, 0, 2)
    v_bt = v_tok.transpose(1, 0, 2)
    cmpk_tok = cmp_k.transpose(0, 2, 1, 3).reshape(B, nb, D)
    cmpk_tok = jnp.concatenate(
        [cmpk_tok, jnp.zeros((B, 128 - nb, D), jnp.float32)], axis=1)
    qmean = q.mean(axis=2).reshape(B, 1, D)

    pos_pad = jnp.zeros((B * S, 128), jnp.float32).at[:, :3].set(
        positions.transpose(1, 0, 2).reshape(B * S, 3))
    w1v = W_c1[:D]
    w1p = jnp.zeros((128, D), jnp.float32).at[:3].set(W_c1[D:])

    halfs = pl.pallas_call(
        _compress1_body,
        grid=(B * S // 128,),
        in_specs=[
            pl.BlockSpec((128, D), lambda i: (i, 0)),
            pl.BlockSpec((128, 128), lambda i: (i, 0)),
            pl.BlockSpec((D, D), lambda i: (0, 0)),
            pl.BlockSpec((128, D), lambda i: (0, 0)),
            pl.BlockSpec((1, D), lambda i: (0, 0)),
        ],
        out_specs=pl.BlockSpec((8, D), lambda i: (i, 0)),
        out_shape=jax.ShapeDtypeStruct((B * S // 16, D), jnp.float32),
    )(v_bt.reshape(B * S, D), pos_pad, w1v, w1p, b_c1.reshape(1, D))

    cmp_v_tok = pl.pallas_call(
        _compress2_body,
        grid=(B,),
        in_specs=[
            pl.BlockSpec((1, 128, D), lambda b: (b, 0, 0)),
            pl.BlockSpec((D, D), lambda b: (0, 0)),
            pl.BlockSpec((1, D), lambda b: (0, 0)),
        ],
        out_specs=pl.BlockSpec((1, 128, D), lambda b: (b, 0, 0)),
        out_shape=jax.ShapeDtypeStruct((B, 128, D), jnp.float32),
    )(halfs.reshape(B, 128, D), W_c2, b_c2.reshape(1, D))

    wg_c = jnp.zeros((8, D), jnp.float32).at[:3].set(W_g[0:D].T)
    wg_s = jnp.zeros((8, D), jnp.float32).at[:3].set(W_g[D:2 * D].T)
    wg_w = jnp.zeros((8, D), jnp.float32).at[:3].set(W_g[2 * D:].T)

    sel_k, sel_v = pl.pallas_call(
        _sel_assemble_body,
        grid=(B,),
        in_specs=[
            pl.BlockSpec(memory_space=pltpu.SMEM),
            pl.BlockSpec((1, S, D), lambda b: (b, 0, 0)),
            pl.BlockSpec((1, S, D), lambda b: (b, 0, 0)),
        ],
        out_specs=[
            pl.BlockSpec((1, NSEL * SELB, D), lambda b: (b, 0, 0)),
            pl.BlockSpec((1, NSEL * SELB, D), lambda b: (b, 0, 0)),
        ],
        out_shape=[
            jax.ShapeDtypeStruct((B, NSEL * SELB, D), jnp.float32),
            jax.ShapeDtypeStruct((B, NSEL * SELB, D), jnp.float32),
        ],
    )(top_blocks, k_bt, v_bt)

    out = pl.pallas_call(
        _attn_gate_body,
        grid=(B, S // SQ),
        in_specs=[
            pl.BlockSpec((1, SQ, D), lambda b, s: (b, s, 0)),
            pl.BlockSpec((1, NSEL * SELB, D), lambda b, s: (b, 0, 0)),
            pl.BlockSpec((1, NSEL * SELB, D), lambda b, s: (b, 0, 0)),
            pl.BlockSpec((1, WIN, D), lambda b, s: (b, S // WIN - 1, 0)),
            pl.BlockSpec((1, WIN, D), lambda b, s: (b, S // WIN - 1, 0)),
            pl.BlockSpec((1, 128, D), lambda b, s: (b, 0, 0)),
            pl.BlockSpec((1, 128, D), lambda b, s: (b, 0, 0)),
            pl.BlockSpec((1, 1, D), lambda b, s: (b, 0, 0)),
            pl.BlockSpec((8, D), lambda b, s: (0, 0)),
            pl.BlockSpec((8, D), lambda b, s: (0, 0)),
            pl.BlockSpec((8, D), lambda b, s: (0, 0)),
            pl.BlockSpec(memory_space=pltpu.SMEM),
        ],
        out_specs=pl.BlockSpec((1, SQ, D), lambda b, s: (b, s, 0)),
        out_shape=jax.ShapeDtypeStruct((B, S, D), jnp.float32),
    )(q.transpose(0, 2, 1, 3).reshape(B, S, D), sel_k, sel_v, k_bt, v_bt,
      cmpk_tok, cmp_v_tok, qmean, wg_c, wg_s, wg_w, b_g)

    return out
